# baseline (speedup 1.0000x reference)
import functools

import jax
import jax.numpy as jnp
from jax import lax
from jax.experimental import pallas as pl
from jax.experimental.pallas import tpu as pltpu
from jax.experimental.pallas import tpu_sc as plsc

MAXLEN = 4096
DIM = 2048
NROWS = 8192

RBLK = 512
NC, NS = 2, 16
NW = NC * NS
WROWS = MAXLEN // NW
CCHUNK = 32


def _reduce_body(x_ref, acc_ref):
    i = pl.program_id(0)

    @pl.when(i == 0)
    def _():
        acc_ref[...] = jnp.zeros_like(acc_ref)

    acc_ref[...] += jnp.sum(x_ref[...], axis=0, keepdims=True)

    @pl.when(i == pl.num_programs(0) - 1)
    def _():
        acc_ref[...] *= (1.0 / NROWS)


def _sc_copy_body(buf_hbm, out_hbm, stage0, stage1, sem0, sem1):
    wid = lax.axis_index("s") * NC + lax.axis_index("c")
    base = wid * WROWS
    stages = (stage0, stage1)
    sems = (sem0, sem1)
    nchunks = WROWS // CCHUNK

    pltpu.make_async_copy(
        buf_hbm.at[pl.ds(base, CCHUNK)], stages[0], sems[0]).start()
    for k in range(nchunks):
        cur, nxt = k % 2, (k + 1) % 2
        if k + 1 < nchunks:
            pltpu.make_async_copy(
                buf_hbm.at[pl.ds(base + (k + 1) * CCHUNK, CCHUNK)],
                stages[nxt], sems[nxt]).start()
        pltpu.make_async_copy(
            buf_hbm.at[pl.ds(base + k * CCHUNK, CCHUNK)],
            stages[cur], sems[cur]).wait()
        pltpu.sync_copy(stages[cur], out_hbm.at[pl.ds(base + k * CCHUNK, CCHUNK)])


def _scatter_body(pos_ref, copied_hbm, mean_ref, out_hbm, sem):
    row = pltpu.make_async_copy(
        mean_ref, out_hbm.at[pl.ds(pos_ref[0], 1), :], sem)
    row.start()
    row.wait()


def kernel(x, buffer, position, length):
    del length
    pos = jnp.asarray(position, jnp.int32).reshape(1)

    mean = pl.pallas_call(
        _reduce_body,
        grid=(NROWS // RBLK,),
        in_specs=[pl.BlockSpec((RBLK, DIM), lambda i: (i, 0))],
        out_specs=pl.BlockSpec((1, DIM), lambda i: (0, 0)),
        out_shape=jax.ShapeDtypeStruct((1, DIM), jnp.float32),
    )(x)

    sc_copy = pl.kernel(
        _sc_copy_body,
        out_type=jax.ShapeDtypeStruct((MAXLEN, DIM), jnp.float32),
        mesh=plsc.VectorSubcoreMesh(
            core_axis_name="c", subcore_axis_name="s",
            num_cores=NC, num_subcores=NS),
        scratch_types=[
            pltpu.VMEM((CCHUNK, DIM), jnp.float32),
            pltpu.VMEM((CCHUNK, DIM), jnp.float32),
            pltpu.SemaphoreType.DMA,
            pltpu.SemaphoreType.DMA,
        ],
    )
    copied = sc_copy(buffer)

    new_buffer = pl.pallas_call(
        _scatter_body,
        grid_spec=pltpu.PrefetchScalarGridSpec(
            num_scalar_prefetch=1,
            grid=(1,),
            in_specs=[
                pl.BlockSpec(memory_space=pltpu.MemorySpace.HBM),
                pl.BlockSpec((1, DIM), lambda i, p: (0, 0)),
            ],
            out_specs=pl.BlockSpec(memory_space=pltpu.MemorySpace.HBM),
            scratch_shapes=[pltpu.SemaphoreType.DMA],
        ),
        out_shape=jax.ShapeDtypeStruct((MAXLEN, DIM), jnp.float32),
        input_output_aliases={1: 0},
    )(pos, copied, mean)

    return new_buffer


# speedup vs baseline: 1.4505x; 1.4505x over previous
import jax
import jax.numpy as jnp
from jax.experimental import pallas as pl
from jax.experimental.pallas import tpu as pltpu

MAXLEN = 4096
DIM = 2048
NROWS = 8192

GRID = 8
RBLK = NROWS // GRID
CBLK = MAXLEN // GRID


def _perm(i, pos_ref):
    b_pos = pos_ref[0] // CBLK
    return jnp.where(i == GRID - 1, b_pos, i + (i >= b_pos).astype(i.dtype))


def _body(pos_ref, x_ref, buf_ref, out_ref, acc_ref):
    i = pl.program_id(0)

    @pl.when(i == 0)
    def _():
        acc_ref[...] = jnp.zeros_like(acc_ref)

    acc_ref[...] += jnp.sum(x_ref[...], axis=0, keepdims=True)
    out_ref[...] = buf_ref[...]

    @pl.when(i == GRID - 1)
    def _():
        local = pos_ref[0] % CBLK
        out_ref[pl.ds(local, 1), :] = acc_ref[...] * (1.0 / NROWS)


def kernel(x, buffer, position, length):
    del length
    pos = jnp.asarray(position, jnp.int32).reshape(1)

    new_buffer = pl.pallas_call(
        _body,
        grid_spec=pltpu.PrefetchScalarGridSpec(
            num_scalar_prefetch=1,
            grid=(GRID,),
            in_specs=[
                pl.BlockSpec((RBLK, DIM), lambda i, p: (i, 0)),
                pl.BlockSpec((CBLK, DIM), lambda i, p: (_perm(i, p), 0)),
            ],
            out_specs=pl.BlockSpec((CBLK, DIM), lambda i, p: (_perm(i, p), 0)),
            scratch_shapes=[pltpu.VMEM((1, DIM), jnp.float32)],
        ),
        out_shape=jax.ShapeDtypeStruct((MAXLEN, DIM), jnp.float32),
    )(pos, x, buffer)

    return new_buffer
